# baseline (speedup 1.0000x reference)
import functools

import jax
import jax.numpy as jnp
from jax import lax
from jax.experimental import pallas as pl
from jax.experimental.pallas import tpu as pltpu
from jax.experimental.pallas import tpu_sc as plsc


def _make_argmax_body(row_stride, act_stride):
    def body(x_ref, o_ref):
        v = x_ref[...]
        m = jnp.max(v, axis=-1, keepdims=True)
        col = lax.broadcasted_iota(jnp.int32, v.shape, 2)
        am = jnp.min(jnp.where(v == m, col, jnp.int32(2**30)), axis=-1)
        pos = lax.broadcasted_iota(jnp.int32, am.shape, 1)
        coeff = jnp.where(pos == 0, row_stride, act_stride)
        o_ref[0, 0, :] = jnp.sum(am * coeff, axis=1)

    return body


def _tc_argmax(s, block_rows, row_stride, act_stride):
    batch, two, cols = s.shape
    grid = batch // block_rows
    out = pl.pallas_call(
        _make_argmax_body(row_stride, act_stride),
        grid=(grid,),
        in_specs=[pl.BlockSpec((block_rows, two, cols), lambda i: (i, 0, 0))],
        out_specs=pl.BlockSpec((1, 1, block_rows), lambda i: (i, 0, 0)),
        out_shape=jax.ShapeDtypeStruct((grid, 1, block_rows), jnp.int32),
    )(s)
    return out.reshape(batch)


_CHUNK = 128


def _make_sc_gather(batch):
    info = plsc.get_sparse_core_info()
    nc, ns, L = info.num_cores, info.num_subcores, info.num_lanes
    nw = nc * ns
    bpw = batch // nw
    nchunk = bpw // _CHUNK
    mesh = plsc.VectorSubcoreMesh(core_axis_name="c", subcore_axis_name="s")

    @functools.partial(
        pl.kernel,
        mesh=mesh,
        out_type=jax.ShapeDtypeStruct((batch,), jnp.float32),
        scratch_types=[
            pltpu.VMEM((nchunk, _CHUNK), jnp.int32),
            pltpu.VMEM((nchunk, _CHUNK), jnp.int32),
            pltpu.VMEM((nchunk, _CHUNK), jnp.float32),
            pltpu.SemaphoreType.DMA,
        ],
    )
    def sc_gather(comb_hbm, a_hbm, table_hbm, out_hbm, idx_v, a_v, out_v, sem):
        wid = lax.axis_index("s") * nc + lax.axis_index("c")
        base = wid * bpw
        for c in range(nchunk):
            pltpu.sync_copy(comb_hbm.at[pl.ds(base + c * _CHUNK, _CHUNK)], idx_v.at[c])
            pltpu.sync_copy(a_hbm.at[pl.ds(base + c * _CHUNK, _CHUNK)], a_v.at[c])
        for c in range(nchunk):
            for o in range(_CHUNK // L):
                sl = pl.ds(o * L, L)
                idx_v[c, sl] = idx_v[c, sl] + a_v[c, sl]
        copies = [
            pltpu.async_copy(table_hbm.at[idx_v.at[c]], out_v.at[c], sem)
            for c in range(nchunk)
        ]
        for cp in copies:
            cp.wait()
        for c in range(nchunk):
            pltpu.sync_copy(out_v.at[c], out_hbm.at[pl.ds(base + c * _CHUNK, _CHUNK)])

    return sc_gather


def kernel(s, a, env_size, table):
    batch = s.shape[0]
    e = s.shape[2]
    acts = table.shape[2]
    comb = _tc_argmax(s, 1024, e * acts, acts)
    a32 = a.astype(jnp.int32)
    tflat = table.reshape(-1)
    sc_gather = _make_sc_gather(batch)
    return sc_gather(comb, a32, tflat)


# speedup vs baseline: 16.7768x; 16.7768x over previous
import functools

import jax
import jax.numpy as jnp
from jax import lax
from jax.experimental import pallas as pl
from jax.experimental.pallas import tpu as pltpu
from jax.experimental.pallas import tpu_sc as plsc


def _make_argmax_body(row_stride):
    def body(x_ref, o_ref):
        v = x_ref[...]
        m = jnp.max(v, axis=1, keepdims=True)
        col = lax.broadcasted_iota(jnp.int32, v.shape, 1)
        am = jnp.min(jnp.where(v == m, col, jnp.int32(2**30)), axis=1)
        o_ref[0, 0, :] = am[0, :] * row_stride + am[1, :]

    return body


def _tc_argmax(st, block_cols, row_stride):
    two, cols, batch = st.shape
    grid = batch // block_cols
    out = pl.pallas_call(
        _make_argmax_body(row_stride),
        grid=(grid,),
        in_specs=[pl.BlockSpec((two, cols, block_cols), lambda i: (0, 0, i))],
        out_specs=pl.BlockSpec((1, 1, block_cols), lambda i: (i, 0, 0)),
        out_shape=jax.ShapeDtypeStruct((grid, 1, block_cols), jnp.int32),
    )(st)
    return out.reshape(batch)


_CHUNK = 128


def _make_sc_gather(batch, row_elems):
    info = plsc.get_sparse_core_info()
    nc, ns, L = info.num_cores, info.num_subcores, info.num_lanes
    nw = nc * ns
    bpw = batch // nw
    nchunk = bpw // _CHUNK
    mesh = plsc.VectorSubcoreMesh(core_axis_name="c", subcore_axis_name="s")

    @functools.partial(
        pl.kernel,
        mesh=mesh,
        out_type=jax.ShapeDtypeStruct((batch,), jnp.float32),
        scratch_types=[
            pltpu.VMEM((nchunk, _CHUNK), jnp.int32),
            pltpu.VMEM((nchunk, _CHUNK), jnp.int32),
            pltpu.VMEM((nchunk, _CHUNK), jnp.float32),
            pltpu.SemaphoreType.DMA,
        ],
        compiler_params=pltpu.CompilerParams(use_tc_tiling_on_sc=False),
    )
    def sc_gather(comb_hbm, a_hbm, table_hbm, out_hbm, idx_v, a_v, out_v, sem):
        wid = lax.axis_index("s") * nc + lax.axis_index("c")
        base = wid * bpw
        for c in range(nchunk):
            pltpu.sync_copy(comb_hbm.at[pl.ds(base + c * _CHUNK, _CHUNK)], idx_v.at[c])
            pltpu.sync_copy(a_hbm.at[pl.ds(base + c * _CHUNK, _CHUNK)], a_v.at[c])
        for c in range(nchunk):
            for o in range(_CHUNK // L):
                sl = pl.ds(o * L, L)
                idx_v[c, sl] = idx_v[c, sl] + a_v[c, sl] * row_elems
        copies = [
            pltpu.async_copy(table_hbm.at[idx_v.at[c]], out_v.at[c], sem)
            for c in range(nchunk)
        ]
        for cp in copies:
            cp.wait()
        for c in range(nchunk):
            pltpu.sync_copy(out_v.at[c], out_hbm.at[pl.ds(base + c * _CHUNK, _CHUNK)])

    return sc_gather


def kernel(s, a, env_size, table):
    batch = s.shape[0]
    e = s.shape[2]
    acts = table.shape[2]
    st = s.transpose(1, 2, 0)
    comb = _tc_argmax(st, 1024, e * acts)
    a32 = a.astype(jnp.int32)
    tflat = table.transpose(0, 2, 1).reshape(-1)
    sc_gather = _make_sc_gather(batch, e)
    return sc_gather(comb, a32, tflat)
